# Initial kernel scaffold; baseline (speedup 1.0000x reference)
import jax
import jax.numpy as jnp
from jax.experimental import pallas as pl


def kernel(feat, adj, ppmi, pl_W1, pl_b1, pl_W2, pl_b2, pl_W3, pl_b3, pg_W1, pg_b1, pg_W2, pg_b2, pg_W3, pg_b3, sl_W1, sl_b1, sl_W2, sl_b2, sl_W3, sl_b3, sg_W1, sg_b1, sg_W2, sg_b2, sg_W3, sg_b3, att_W, att_b):
    raise NotImplementedError("write your pallas kernel here")


# speedup vs baseline: 2.4014x; 2.4014x over previous
import functools

import jax
import jax.numpy as jnp
from jax.experimental import pallas as pl
from jax.experimental.pallas import tpu as pltpu

_BM = 400
_BP = 2000


def _spmm_kernel(relu_cols, out_dtype, a_ref, b_ref, bias_ref, o_ref):
    r = jnp.dot(a_ref[...].astype(jnp.bfloat16), b_ref[...],
                preferred_element_type=jnp.float32) + bias_ref[...]
    if relu_cols is not None:
        lo, hi = relu_cols
        col = jax.lax.broadcasted_iota(jnp.int32, r.shape, 1)
        r = jnp.where((col >= lo) & (col < hi), jnp.maximum(r, 0.0), r)
    o_ref[...] = r.astype(out_dtype)


def _spmm(a, b, bias, relu_cols, out_dtype):
    n = a.shape[0]
    c = b.shape[1]
    return pl.pallas_call(
        functools.partial(_spmm_kernel, relu_cols, out_dtype),
        grid=(n // _BM,),
        in_specs=[
            pl.BlockSpec((_BM, n), lambda i: (i, 0)),
            pl.BlockSpec((n, c), lambda i: (0, 0)),
            pl.BlockSpec((1, c), lambda i: (0, 0)),
        ],
        out_specs=pl.BlockSpec((_BM, c), lambda i: (i, 0)),
        out_shape=jax.ShapeDtypeStruct((n, c), out_dtype),
        compiler_params=pltpu.CompilerParams(
            dimension_semantics=("arbitrary",)),
    )(a, b, bias)


def _proj_kernel(x_ref, w_ref, o_ref):
    o_ref[...] = jnp.dot(x_ref[...], w_ref[...],
                         preferred_element_type=jnp.float32
                         ).astype(jnp.bfloat16)


def _proj(x, w):
    n, kdim = x.shape
    c = w.shape[1]
    return pl.pallas_call(
        _proj_kernel,
        grid=(n // _BP,),
        in_specs=[pl.BlockSpec((_BP, kdim), lambda i: (i, 0)),
                  pl.BlockSpec((kdim, c), lambda i: (0, 0))],
        out_specs=pl.BlockSpec((_BP, c), lambda i: (i, 0)),
        out_shape=jax.ShapeDtypeStruct((n, c), jnp.bfloat16),
        compiler_params=pltpu.CompilerParams(
            dimension_semantics=("parallel",)),
    )(x, w)


def _att_kernel(m1_ref, m2_ref, w_ref, b_ref, o_ref):
    m1 = m1_ref[...]
    m2 = m2_ref[...]
    w = w_ref[...]
    b = b_ref[0, 0]
    l1 = jnp.dot(m1, w, preferred_element_type=jnp.float32) + b
    l2 = jnp.dot(m2, w, preferred_element_type=jnp.float32) + b
    z = jnp.maximum(l1, l2)
    e1 = jnp.exp(l1 - z)
    e2 = jnp.exp(l2 - z)
    o_ref[...] = (e1 * m1 + e2 * m2) / (e1 + e2)


def _attention(m1, m2, att_w, att_b):
    n, c = m1.shape
    return pl.pallas_call(
        _att_kernel,
        grid=(n // _BP,),
        in_specs=[pl.BlockSpec((_BP, c), lambda i: (i, 0)),
                  pl.BlockSpec((_BP, c), lambda i: (i, 0)),
                  pl.BlockSpec((c, 1), lambda i: (0, 0)),
                  pl.BlockSpec((1, 1), lambda i: (0, 0))],
        out_specs=pl.BlockSpec((_BP, c), lambda i: (i, 0)),
        out_shape=jax.ShapeDtypeStruct((n, c), jnp.float32),
        compiler_params=pltpu.CompilerParams(
            dimension_semantics=("parallel",)),
    )(m1, m2, att_w, att_b)


def _block_diag(w_top, w_bot):
    top = jnp.concatenate([w_top, jnp.zeros_like(w_bot)], axis=1)
    bot = jnp.concatenate([jnp.zeros_like(w_top), w_bot], axis=1)
    return jnp.concatenate([top, bot], axis=0)


def kernel(feat, adj, ppmi,
           pl_W1, pl_b1, pl_W2, pl_b2, pl_W3, pl_b3,
           pg_W1, pg_b1, pg_W2, pg_b2, pg_W3, pg_b3,
           sl_W1, sl_b1, sl_W2, sl_b2, sl_W3, sl_b3,
           sg_W1, sg_b1, sg_W2, sg_b2, sg_W3, sg_b3,
           att_W, att_b):
    bf = jnp.bfloat16
    hid = pl_W1.shape[1]
    out = pl_W2.shape[1]
    featb = feat.astype(bf)

    w1_adj = jnp.concatenate([pl_W1, sl_W1], axis=1).astype(bf)
    w1_ppmi = jnp.concatenate([pg_W1, sg_W1], axis=1).astype(bf)
    b1_adj = jnp.concatenate([pl_b1, sl_b1]).reshape(1, -1)
    b1_ppmi = jnp.concatenate([pg_b1, sg_b1]).reshape(1, -1)

    s_adj = _spmm(adj, _proj(featb, w1_adj), b1_adj,
                  relu_cols=(hid, 2 * hid), out_dtype=bf)
    s_ppmi = _spmm(ppmi, _proj(featb, w1_ppmi), b1_ppmi,
                   relu_cols=(hid, 2 * hid), out_dtype=bf)

    wd_adj = _block_diag(jnp.concatenate([pl_W2, pl_W3], axis=1),
                         jnp.concatenate([sl_W2, sl_W3], axis=1)).astype(bf)
    wd_ppmi = _block_diag(jnp.concatenate([pg_W2, pg_W3], axis=1),
                          jnp.concatenate([sg_W2, sg_W3], axis=1)).astype(bf)
    b2_adj = jnp.concatenate([pl_b2, pl_b3, sl_b2, sl_b3]).reshape(1, -1)
    b2_ppmi = jnp.concatenate([pg_b2, pg_b3, sg_b2, sg_b3]).reshape(1, -1)

    o_adj = _spmm(adj, _proj(s_adj, wd_adj), b2_adj,
                  relu_cols=None, out_dtype=jnp.float32)
    o_ppmi = _spmm(ppmi, _proj(s_ppmi, wd_ppmi), b2_ppmi,
                   relu_cols=None, out_dtype=jnp.float32)

    mu_p_l = o_adj[:, :out]
    logvar_p_l = o_adj[:, out:2 * out]
    mu_s_l = o_adj[:, 2 * out:3 * out]
    logvar_s_l = o_adj[:, 3 * out:]
    mu_p_g = o_ppmi[:, :out]
    logvar_p_g = o_ppmi[:, out:2 * out]
    mu_s_g = o_ppmi[:, 2 * out:3 * out]
    logvar_s_g = o_ppmi[:, 3 * out:]

    shared_emb = _attention(mu_s_l, mu_s_g, att_W, att_b.reshape(1, 1))

    return (mu_p_l, mu_p_l, logvar_p_l,
            mu_p_g, mu_p_g, logvar_p_g,
            mu_s_l, mu_s_l, logvar_s_l,
            mu_s_g, mu_s_g, logvar_s_g,
            shared_emb)
